# baseline (speedup 1.0000x reference)
import functools

import numpy as np
import jax
import jax.numpy as jnp
from jax import lax
from jax.experimental import pallas as pl
from jax.experimental.pallas import tpu as pltpu
from jax.experimental.pallas import tpu_sc as plsc

_B = 16384
_N_CAT = 26
_N_CONT = 13
_VOCAB1 = 100001
_DIM = 32
_CCHUNK = 4096
_NCHUNK = _B // _CCHUNK
_PHASE = 13


def _sc_cat_gather(tabt, idxt):
    mesh = plsc.VectorSubcoreMesh(core_axis_name="c", subcore_axis_name="s")
    nc = mesh.num_cores

    @functools.partial(
        pl.kernel,
        out_type=jax.ShapeDtypeStruct((_N_CAT * _DIM, _B), jnp.float32),
        mesh=mesh,
        scratch_types=[
            pltpu.VMEM((_VOCAB1,), jnp.float32),
            pltpu.VMEM((1, _CCHUNK), jnp.int32),
            pltpu.VMEM((1, _CCHUNK), jnp.int32),
            pltpu.VMEM((_CCHUNK,), jnp.float32),
            pltpu.VMEM((_CCHUNK,), jnp.float32),
            pltpu.SemaphoreType.DMA,
            pltpu.SemaphoreType.DMA,
            pltpu.SemaphoreType.DMA,
        ],
        compiler_params=pltpu.CompilerParams(needs_layout_passes=False),
    )
    def k(tab_hbm, idxt_hbm, out_hbm, trow, i0, i1, o0, o1, rsem, isem, osem):
        d = lax.axis_index("s") * nc + lax.axis_index("c")
        ib, ob = (i0, i1), (o0, o1)
        out_pend = [None, None]
        idx_pend = [None, None]

        def fire_idx(i, c):
            idx_pend[c % 2] = pltpu.async_copy(
                idxt_hbm.at[pl.ds(i, 1), pl.ds(c * _CCHUNK, _CCHUNK)],
                ib[c % 2],
                isem,
            )

        fire_idx(0, 0)
        for i in range(_N_CAT):
            row_dma = pltpu.async_copy(tab_hbm.at[i, d], trow, rsem)
            row_dma.wait()
            for c in range(_NCHUNK):
                s = c % 2
                idx_pend[s].wait()
                if c + 1 < _NCHUNK:
                    fire_idx(i, c + 1)
                elif i + 1 < _N_CAT:
                    fire_idx(i + 1, 0)
                if out_pend[s] is not None:
                    out_pend[s].wait()

                @plsc.parallel_loop(0, _CCHUNK // 16, 1, unroll=8)
                def _(w):
                    q = w * 16
                    vals = ib[s][0, pl.ds(q, 16)]
                    ob[s][pl.ds(q, 16)] = plsc.load_gather(trow, [vals])

                out_pend[s] = pltpu.async_copy(
                    ob[s],
                    out_hbm.at[i * _DIM + d, pl.ds(c * _CCHUNK, _CCHUNK)],
                    osem,
                )
        out_pend[0].wait()
        out_pend[1].wait()

    return k(tabt, idxt)


def _cont_body(cont_ref, gam_ref, bet_ref, e_ref, w_ref, b_ref, out_ref):
    x = cont_ref[...]
    mu = jnp.mean(x, axis=0, keepdims=True)
    xc = x - mu
    var = jnp.mean(xc * xc, axis=0, keepdims=True)
    xn = xc * lax.rsqrt(var + 1e-5)
    zg = xn * gam_ref[...] + bet_ref[...]
    ze = jnp.dot(
        e_ref[...],
        zg,
        preferred_element_type=jnp.float32,
        precision=lax.Precision.HIGHEST,
    )
    out_ref[...] = jnp.maximum(ze * w_ref[...] + b_ref[...], 0.0)


_E_CONST = np.repeat(np.eye(_N_CONT, dtype=np.float32), _DIM, axis=0)
_CONT_BLK = 2048
_D_CONT = _N_CONT * _DIM


def _cont_call(cont_t, gam, bet, e, w_flat, b_flat):
    return pl.pallas_call(
        _cont_body,
        grid=(_B // _CONT_BLK,),
        in_specs=[
            pl.BlockSpec((_N_CONT, _CONT_BLK), lambda j: (0, j)),
            pl.BlockSpec((_N_CONT, 1), lambda j: (0, 0)),
            pl.BlockSpec((_N_CONT, 1), lambda j: (0, 0)),
            pl.BlockSpec((_D_CONT, _N_CONT), lambda j: (0, 0)),
            pl.BlockSpec((_D_CONT, 1), lambda j: (0, 0)),
            pl.BlockSpec((_D_CONT, 1), lambda j: (0, 0)),
        ],
        out_specs=pl.BlockSpec((_D_CONT, _CONT_BLK), lambda j: (0, j)),
        out_shape=jax.ShapeDtypeStruct((_D_CONT, _B), jnp.float32),
    )(cont_t, gam, bet, e, w_flat, b_flat)


def kernel(X, emb_tables, cont_weight, cont_bias, ln_gamma, ln_beta):
    xt = X.T
    idxt = xt[:_N_CAT].astype(jnp.int32)
    tabt = jnp.transpose(emb_tables, (0, 2, 1))
    x_cat_t = _sc_cat_gather(tabt, idxt)
    x_cat = x_cat_t.T

    cont_t = xt[_N_CAT:]
    x_cont_t = _cont_call(
        cont_t,
        ln_gamma.reshape(_N_CONT, 1),
        ln_beta.reshape(_N_CONT, 1),
        jnp.asarray(_E_CONST),
        cont_weight.reshape(_D_CONT, 1),
        cont_bias.reshape(_D_CONT, 1),
    )
    return (x_cat, x_cont_t.T)


# speedup vs baseline: 1.0012x; 1.0012x over previous
import functools

import numpy as np
import jax
import jax.numpy as jnp
from jax import lax
from jax.experimental import pallas as pl
from jax.experimental.pallas import tpu as pltpu
from jax.experimental.pallas import tpu_sc as plsc

_B = 16384
_N_CAT = 26
_N_CONT = 13
_VOCAB1 = 100001
_DIM = 32
_CCHUNK = 4096
_NCHUNK = _B // _CCHUNK


def _sc_cat_gather(tabt, idxt):
    mesh = plsc.VectorSubcoreMesh(core_axis_name="c", subcore_axis_name="s")
    nc = mesh.num_cores

    @functools.partial(
        pl.kernel,
        out_type=jax.ShapeDtypeStruct((_N_CAT * _DIM, _B), jnp.float32),
        mesh=mesh,
        scratch_types=[
            pltpu.VMEM((_VOCAB1,), jnp.float32),
            pltpu.VMEM((1, _CCHUNK), jnp.int32),
            pltpu.VMEM((1, _CCHUNK), jnp.int32),
            pltpu.VMEM((_CCHUNK,), jnp.float32),
            pltpu.VMEM((_CCHUNK,), jnp.float32),
            pltpu.SemaphoreType.DMA,
            pltpu.SemaphoreType.DMA,
            pltpu.SemaphoreType.DMA,
        ],
        compiler_params=pltpu.CompilerParams(needs_layout_passes=False),
    )
    def k(tab_hbm, idxt_hbm, out_hbm, trow, i0, i1, o0, o1, rsem, isem, osem):
        d = lax.axis_index("s") * nc + lax.axis_index("c")
        ib, ob = (i0, i1), (o0, o1)
        out_pend = [None, None]
        idx_pend = [None, None]

        def fire_idx(i, c):
            idx_pend[c % 2] = pltpu.async_copy(
                idxt_hbm.at[pl.ds(i, 1), pl.ds(c * _CCHUNK, _CCHUNK)],
                ib[c % 2],
                isem,
            )

        fire_idx(0, 0)
        for i in range(_N_CAT):
            row_dma = pltpu.async_copy(tab_hbm.at[i, d], trow, rsem)
            row_dma.wait()
            for c in range(_NCHUNK):
                s = c % 2
                idx_pend[s].wait()
                if c + 1 < _NCHUNK:
                    fire_idx(i, c + 1)
                elif i + 1 < _N_CAT:
                    fire_idx(i + 1, 0)
                if out_pend[s] is not None:
                    out_pend[s].wait()

                @plsc.parallel_loop(0, _CCHUNK // 16, 1, unroll=8)
                def _(w):
                    q = w * 16
                    vals = ib[s][0, pl.ds(q, 16)]
                    ob[s][pl.ds(q, 16)] = plsc.load_gather(trow, [vals])

                out_pend[s] = pltpu.async_copy(
                    ob[s],
                    out_hbm.at[i * _DIM + d, pl.ds(c * _CCHUNK, _CCHUNK)],
                    osem,
                )
        out_pend[0].wait()
        out_pend[1].wait()

    return k(tabt, idxt)


def _cont_body(cont_ref, gam_ref, bet_ref, e_ref, w_ref, b_ref, out_ref):
    x = cont_ref[...]
    mu = jnp.mean(x, axis=0, keepdims=True)
    xc = x - mu
    var = jnp.mean(xc * xc, axis=0, keepdims=True)
    xn = xc * lax.rsqrt(var + 1e-5)
    zg = xn * gam_ref[...] + bet_ref[...]
    ze = jnp.dot(
        e_ref[...],
        zg,
        preferred_element_type=jnp.float32,
        precision=lax.Precision.HIGHEST,
    )
    out_ref[...] = jnp.maximum(ze * w_ref[...] + b_ref[...], 0.0)


_E_CONST = np.repeat(np.eye(_N_CONT, dtype=np.float32), _DIM, axis=0)
_CONT_BLK = 2048
_D_CONT = _N_CONT * _DIM


def _cont_call(cont_t, gam, bet, e, w_flat, b_flat):
    return pl.pallas_call(
        _cont_body,
        grid=(_B // _CONT_BLK,),
        in_specs=[
            pl.BlockSpec((_N_CONT, _CONT_BLK), lambda j: (0, j)),
            pl.BlockSpec((_N_CONT, 1), lambda j: (0, 0)),
            pl.BlockSpec((_N_CONT, 1), lambda j: (0, 0)),
            pl.BlockSpec((_D_CONT, _N_CONT), lambda j: (0, 0)),
            pl.BlockSpec((_D_CONT, 1), lambda j: (0, 0)),
            pl.BlockSpec((_D_CONT, 1), lambda j: (0, 0)),
        ],
        out_specs=pl.BlockSpec((_D_CONT, _CONT_BLK), lambda j: (0, j)),
        out_shape=jax.ShapeDtypeStruct((_D_CONT, _B), jnp.float32),
    )(cont_t, gam, bet, e, w_flat, b_flat)


def kernel(X, emb_tables, cont_weight, cont_bias, ln_gamma, ln_beta):
    xt = X.T
    idxt = xt[:_N_CAT].astype(jnp.int32)
    tabt = jnp.transpose(emb_tables, (0, 2, 1))
    x_cat_t = _sc_cat_gather(tabt, idxt)
    x_cat = x_cat_t.T

    cont_t = xt[_N_CAT:]
    x_cont_t = _cont_call(
        cont_t,
        ln_gamma.reshape(_N_CONT, 1),
        ln_beta.reshape(_N_CONT, 1),
        jnp.asarray(_E_CONST),
        cont_weight.reshape(_D_CONT, 1),
        cont_bias.reshape(_D_CONT, 1),
    )
    return (x_cat, x_cont_t.T)
